# Initial kernel scaffold; baseline (speedup 1.0000x reference)
import jax
import jax.numpy as jnp
from jax.experimental import pallas as pl


def kernel(h, edge_attr, edge_index, hk_W1, hk_b1, hk_g, hk_be, hk_W2, hk_b2, hv_W1, hv_b1, hv_g, hv_be, hv_W2, hv_b2, hq_W1, hq_b1, hq_g, hq_be, hq_W2, hq_b2, no_W1, no_b1, no_g, no_be, no_W2, no_b2):
    raise NotImplementedError("write your pallas kernel here")


# speedup vs baseline: 33.9798x; 33.9798x over previous
import functools

import jax
import jax.numpy as jnp
import numpy as np
from jax import lax
from jax.experimental import pallas as pl
from jax.experimental.pallas import tpu as pltpu
from jax.experimental.pallas import tpu_sc as plsc

N = 10000
E = 320000
D = 128
H = 16
EF = 16
DH = D // H

NC = 2
NS = 16
NW = NC * NS
EPW = E // NW
CH = 128
NFULL = EPW // CH
TAIL = EPW - NFULL * CH
ZCH = 80
NZ = N // ZCH
EPT2 = 2 * EPW
NFULL2 = EPT2 // CH
TAIL2 = EPT2 - NFULL2 * CH

_S_NP = np.kron(np.eye(H, dtype=np.float32), np.ones((DH, 1), dtype=np.float32))
_B8 = np.kron(np.eye(H // 2, dtype=np.float32), np.ones((1, DH), dtype=np.float32))
_B0_NP = np.vstack([_B8, np.zeros_like(_B8)])
_B1_NP = np.vstack([np.zeros_like(_B8), _B8])
_B0E_NP = np.hstack([_B0_NP, np.eye(H, dtype=np.float32),
                     np.zeros((H, D - D // 2 - H), np.float32)])
_B1E_NP = np.hstack([_B1_NP, np.eye(H, dtype=np.float32),
                     np.zeros((H, D - D // 2 - H), np.float32)])

EB = 2560
NB = 2000


def _ln_relu(t, g, be):
    mu = jnp.mean(t, axis=-1, keepdims=True)
    var = jnp.mean((t - mu) * (t - mu), axis=-1, keepdims=True)
    return jax.nn.relu((t - mu) * lax.rsqrt(var + 1e-5) * g + be)


def _q_body(h_ref, w1, b1, g, be, w2, b2, q_ref):
    t = jnp.dot(h_ref[...], w1[...], preferred_element_type=jnp.float32) + b1[...]
    t = _ln_relu(t, g[...], be[...])
    q_ref[...] = jnp.dot(t, w2[...], preferred_element_type=jnp.float32) + b2[...]


def _tc_q(h, w1, b1, g, be, w2, b2):
    full = lambda shape: pl.BlockSpec(shape, lambda i: (0, 0))
    return pl.pallas_call(
        _q_body,
        grid=(N // NB,),
        in_specs=[
            pl.BlockSpec((NB, D), lambda i: (i, 0)),
            full((D, D)), full((1, D)), full((1, D)), full((1, D)),
            full((D, D)), full((1, D)),
        ],
        out_specs=pl.BlockSpec((NB, D), lambda i: (i, 0)),
        out_shape=jax.ShapeDtypeStruct((N, D), jnp.float32),
    )(h, w1, b1, g, be, w2, b2)


def _sc_gather_body(h_hbm, q_hbm, src_hbm, dst_hbm, hs_out, qd_out,
                    idx1, idx2, rows1, rows2, idx1t, idx2t, rows1t, rows2t,
                    sem1, sem2):
    c = lax.axis_index("c")
    s = lax.axis_index("s")
    w = c * NS + s
    ebase = w * EPW

    def chunk(base, idxa, idxb, ra, rb, n):
        pltpu.sync_copy(src_hbm.at[pl.ds(base, n)], idxa)
        pltpu.sync_copy(dst_hbm.at[pl.ds(base, n)], idxb)
        cp1 = pltpu.async_copy(h_hbm.at[idxa], ra, sem1)
        cp2 = pltpu.async_copy(q_hbm.at[idxb], rb, sem2)
        cp1.wait()
        cp2.wait()
        pltpu.sync_copy(ra, hs_out.at[pl.ds(base, n)])
        pltpu.sync_copy(rb, qd_out.at[pl.ds(base, n)])

    def body(j, carry):
        chunk(ebase + j * CH, idx1, idx2, rows1, rows2, CH)
        return carry

    lax.fori_loop(0, NFULL, body, 0)
    chunk(ebase + NFULL * CH, idx1t, idx2t, rows1t, rows2t, TAIL)


def _sc_gather(h, q, src, dst):
    mesh = plsc.VectorSubcoreMesh(core_axis_name="c", subcore_axis_name="s")
    f = functools.partial(
        pl.kernel,
        out_type=[
            jax.ShapeDtypeStruct((E, D), jnp.float32),
            jax.ShapeDtypeStruct((E, D), jnp.float32),
        ],
        mesh=mesh,
        scratch_types=[
            pltpu.VMEM((CH,), jnp.int32),
            pltpu.VMEM((CH,), jnp.int32),
            pltpu.VMEM((CH, D), jnp.float32),
            pltpu.VMEM((CH, D), jnp.float32),
            pltpu.VMEM((TAIL,), jnp.int32),
            pltpu.VMEM((TAIL,), jnp.int32),
            pltpu.VMEM((TAIL, D), jnp.float32),
            pltpu.VMEM((TAIL, D), jnp.float32),
            pltpu.SemaphoreType.DMA,
            pltpu.SemaphoreType.DMA,
        ],
    )(_sc_gather_body)
    return f(h, q, src, dst)


def _edge_body(hs, qd, ea, kw1h, kw1e, kb1, kg, kbe, kw2, kb2,
               vw1h, vw1e, vb1, vg, vbe, vw2a, vw2b, vb2a, vb2b, s_ref,
               b0_ref, b1_ref, p0_ref, p1_ref):
    hsv = hs[...]
    eav = ea[...]
    tk = (jnp.dot(hsv, kw1h[...], preferred_element_type=jnp.float32)
          + jnp.dot(eav, kw1e[...], preferred_element_type=jnp.float32) + kb1[...])
    kk = jnp.dot(_ln_relu(tk, kg[...], kbe[...]), kw2[...],
                 preferred_element_type=jnp.float32) + kb2[...]
    tv = (jnp.dot(hsv, vw1h[...], preferred_element_type=jnp.float32)
          + jnp.dot(eav, vw1e[...], preferred_element_type=jnp.float32) + vb1[...])
    rv = _ln_relu(tv, vg[...], vbe[...])
    a0 = jnp.dot(rv, vw2a[...], preferred_element_type=jnp.float32) + vb2a[...]
    a1 = jnp.dot(rv, vw2b[...], preferred_element_type=jnp.float32) + vb2b[...]
    logits = jnp.dot(qd[...] * kk, s_ref[...],
                     preferred_element_type=jnp.float32) * (1.0 / np.sqrt(DH))
    ex = jnp.exp(logits)
    p0_ref[...] = a0 * jnp.dot(ex, b0_ref[...], preferred_element_type=jnp.float32)
    p1_ref[...] = a1 * jnp.dot(ex, b1_ref[...], preferred_element_type=jnp.float32)


def _tc_edge(hs, qd, ea, kw1h, kw1e, kb1, kg, kbe, kw2, kb2,
             vw1h, vw1e, vb1, vg, vbe, vw2a, vw2b, vb2a, vb2b,
             s_mat, b0_mat, b1_mat):
    full = lambda shape: pl.BlockSpec(shape, lambda i: (0, 0))
    return pl.pallas_call(
        _edge_body,
        grid=(E // EB,),
        in_specs=[
            pl.BlockSpec((EB, D), lambda i: (i, 0)),
            pl.BlockSpec((EB, D), lambda i: (i, 0)),
            pl.BlockSpec((EB, EF), lambda i: (i, 0)),
            full((D, D)), full((EF, D)), full((1, D)), full((1, D)), full((1, D)),
            full((D, D)), full((1, D)),
            full((D, D)), full((EF, D)), full((1, D)), full((1, D)), full((1, D)),
            full((D, D)), full((D, D)), full((1, D)), full((1, D)),
            full((D, H)), full((H, D)), full((H, D)),
        ],
        out_specs=[
            pl.BlockSpec((EB, D), lambda i: (i, 0)),
            pl.BlockSpec((EB, D), lambda i: (i, 0)),
        ],
        out_shape=[
            jax.ShapeDtypeStruct((E, D), jnp.float32),
            jax.ShapeDtypeStruct((E, D), jnp.float32),
        ],
    )(hs, qd, ea, kw1h, kw1e, kb1, kg, kbe, kw2, kb2,
      vw1h, vw1e, vb1, vg, vbe, vw2a, vw2b, vb2a, vb2b, s_mat, b0_mat, b1_mat)


def _sc_scatter_body(p0_hbm, p1_hbm, dst_hbm, zp_hbm, acc_out,
                     acc_sh, idxr, pbuf, idxt, pbuft, rbuf):
    c = lax.axis_index("c")
    s = lax.axis_index("s")

    nz_mine = (NZ - s + NS - 1) // NS
    pltpu.sync_copy(zp_hbm, rbuf)

    def zbody(i, carry):
        r0 = (s + i * NS) * ZCH
        pltpu.sync_copy(rbuf, acc_sh.at[pl.ds(r0, ZCH)])
        return carry

    lax.fori_loop(0, nz_mine, zbody, 0)
    plsc.subcore_barrier()

    abase = s * EPT2

    def achunk(p_hbm, base, idxa, pb, n):
        pltpu.sync_copy(dst_hbm.at[pl.ds(base, n)], idxa)
        pltpu.sync_copy(p_hbm.at[pl.ds(base, n)], pb)
        pltpu.sync_copy(pb, acc_sh.at[idxa], add=True)

    def run_acc(p_hbm):
        def body(j, carry):
            achunk(p_hbm, abase + j * CH, idxr, pbuf, CH)
            return carry
        lax.fori_loop(0, NFULL2, body, 0)
        achunk(p_hbm, abase + NFULL2 * CH, idxt, pbuft, TAIL2)

    @pl.when(c == 0)
    def _():
        run_acc(p0_hbm)

    @pl.when(c == 1)
    def _():
        run_acc(p1_hbm)

    plsc.subcore_barrier()

    def fbody(i, carry):
        r0 = (s + i * NS) * ZCH
        pltpu.sync_copy(acc_sh.at[pl.ds(r0, ZCH)], rbuf)
        pltpu.sync_copy(rbuf, acc_out.at[c, pl.ds(r0, ZCH)])
        return carry

    lax.fori_loop(0, nz_mine, fbody, 0)


def _sc_scatter(p0, p1, dst, zp):
    mesh = plsc.VectorSubcoreMesh(core_axis_name="c", subcore_axis_name="s")
    f = functools.partial(
        pl.kernel,
        out_type=jax.ShapeDtypeStruct((NC, N, D), jnp.float32),
        mesh=mesh,
        scratch_types=[
            pltpu.VMEM_SHARED((N, D), jnp.float32),
            pltpu.VMEM((CH,), jnp.int32),
            pltpu.VMEM((CH, D), jnp.float32),
            pltpu.VMEM((TAIL2,), jnp.int32),
            pltpu.VMEM((TAIL2, D), jnp.float32),
            pltpu.VMEM((ZCH, D), jnp.float32),
        ],
    )(_sc_scatter_body)
    return f(p0, p1, dst, zp)


def _out_body(a0, a1, d0, d1, h_ref, b0_ref, b1_ref,
              w1a0, w1a1, w1b, b1, g, be, w2, b2, o_ref):
    oa0 = a0[...] / (jnp.dot(d0[...], b0_ref[...], preferred_element_type=jnp.float32) + 1e-16)
    oa1 = a1[...] / (jnp.dot(d1[...], b1_ref[...], preferred_element_type=jnp.float32) + 1e-16)
    hv = h_ref[...]
    t = (jnp.dot(oa0, w1a0[...], preferred_element_type=jnp.float32)
         + jnp.dot(oa1, w1a1[...], preferred_element_type=jnp.float32)
         + jnp.dot(hv, w1b[...], preferred_element_type=jnp.float32) + b1[...])
    t = _ln_relu(t, g[...], be[...])
    o_ref[...] = jnp.dot(t, w2[...], preferred_element_type=jnp.float32) + b2[...] + hv


def _tc_out(a0, a1, d0, d1, h, b0_mat, b1_mat, w1a0, w1a1, w1b, b1, g, be, w2, b2):
    full = lambda shape: pl.BlockSpec(shape, lambda i: (0, 0))
    return pl.pallas_call(
        _out_body,
        grid=(N // NB,),
        in_specs=[
            pl.BlockSpec((NB, D // 2), lambda i: (i, 0)),
            pl.BlockSpec((NB, D // 2), lambda i: (i, 0)),
            pl.BlockSpec((NB, H), lambda i: (i, 0)),
            pl.BlockSpec((NB, H), lambda i: (i, 0)),
            pl.BlockSpec((NB, D), lambda i: (i, 0)),
            full((H, D // 2)), full((H, D // 2)),
            full((D // 2, D)), full((D // 2, D)), full((D, D)),
            full((1, D)), full((1, D)), full((1, D)),
            full((D, D)), full((1, D)),
        ],
        out_specs=pl.BlockSpec((NB, D), lambda i: (i, 0)),
        out_shape=jax.ShapeDtypeStruct((N, D), jnp.float32),
    )(a0, a1, d0, d1, h, b0_mat, b1_mat, w1a0, w1a1, w1b, b1, g, be, w2, b2)


def kernel(h, edge_attr, edge_index,
           hk_W1, hk_b1, hk_g, hk_be, hk_W2, hk_b2,
           hv_W1, hv_b1, hv_g, hv_be, hv_W2, hv_b2,
           hq_W1, hq_b1, hq_g, hq_be, hq_W2, hq_b2,
           no_W1, no_b1, no_g, no_be, no_W2, no_b2):
    src = edge_index[0]
    dst = edge_index[1]
    r = lambda x: x.reshape(1, D)

    s_mat = jnp.asarray(_S_NP)
    b0_mat = jnp.asarray(_B0_NP)
    b1_mat = jnp.asarray(_B1_NP)
    b0e_mat = jnp.asarray(_B0E_NP)
    b1e_mat = jnp.asarray(_B1E_NP)
    half = D // 2
    zcol = jnp.zeros((D, half), jnp.float32)
    ebias = jnp.concatenate([jnp.ones((H,), jnp.float32),
                             jnp.zeros((half - H,), jnp.float32)])
    vw2a = jnp.concatenate([hv_W2[:, :half], zcol], axis=1)
    vw2b = jnp.concatenate([hv_W2[:, half:], zcol], axis=1)
    vb2a = jnp.concatenate([hv_b2[:half], ebias]).reshape(1, D)
    vb2b = jnp.concatenate([hv_b2[half:], ebias]).reshape(1, D)

    q = _tc_q(h, hq_W1, r(hq_b1), r(hq_g), r(hq_be), hq_W2, r(hq_b2))
    hs, qd = _sc_gather(h, q, src, dst)
    p0, p1 = _tc_edge(
        hs, qd, edge_attr,
        hk_W1[:D], hk_W1[D:], r(hk_b1), r(hk_g), r(hk_be), hk_W2, r(hk_b2),
        hv_W1[:D], hv_W1[D:], r(hv_b1), r(hv_g), r(hv_be),
        vw2a, vw2b, vb2a, vb2b,
        s_mat, b0e_mat, b1e_mat)
    zp = jnp.zeros((ZCH, D), jnp.float32)
    acc = _sc_scatter(p0, p1, dst, zp)
    a0 = acc[0, :, :half]
    d0 = acc[0, :, half:half + H]
    a1 = acc[1, :, :half]
    d1 = acc[1, :, half:half + H]
    out = _tc_out(a0, a1, d0, d1, h, b0_mat, b1_mat,
                  no_W1[:half], no_W1[half:D], no_W1[D:],
                  r(no_b1), r(no_g), r(no_be), no_W2, r(no_b2))
    return out


# speedup vs baseline: 40.0642x; 1.1791x over previous
import functools

import jax
import jax.numpy as jnp
import numpy as np
from jax import lax
from jax.experimental import pallas as pl
from jax.experimental.pallas import tpu as pltpu
from jax.experimental.pallas import tpu_sc as plsc

N = 10000
E = 320000
D = 128
H = 16
EF = 16
DH = D // H

NC = 2
NS = 16
NW = NC * NS
EPW = E // NW
CH = 128
NFULL = EPW // CH
TAIL = EPW - NFULL * CH
ZCH = 80
NZ = N // ZCH
EPT2 = 2 * EPW
NFULL2 = EPT2 // CH
TAIL2 = EPT2 - NFULL2 * CH

_S_NP = np.kron(np.eye(H, dtype=np.float32), np.ones((DH, 1), dtype=np.float32))
_B8 = np.kron(np.eye(H // 2, dtype=np.float32), np.ones((1, DH), dtype=np.float32))
_B0_NP = np.vstack([_B8, np.zeros_like(_B8)])
_B1_NP = np.vstack([np.zeros_like(_B8), _B8])
_B0E_NP = np.hstack([_B0_NP, np.eye(H, dtype=np.float32),
                     np.zeros((H, D - D // 2 - H), np.float32)])
_B1E_NP = np.hstack([_B1_NP, np.eye(H, dtype=np.float32),
                     np.zeros((H, D - D // 2 - H), np.float32)])

EB = 2560
NB = 2000


def _ln_relu(t, g, be):
    mu = jnp.mean(t, axis=-1, keepdims=True)
    var = jnp.mean((t - mu) * (t - mu), axis=-1, keepdims=True)
    return jax.nn.relu((t - mu) * lax.rsqrt(var + 1e-5) * g + be)


def _q_body(h_ref, w1, b1, g, be, w2, b2, q_ref):
    t = jnp.dot(h_ref[...], w1[...], preferred_element_type=jnp.float32) + b1[...]
    t = _ln_relu(t, g[...], be[...])
    q_ref[...] = jnp.dot(t, w2[...], preferred_element_type=jnp.float32) + b2[...]


def _tc_q(h, w1, b1, g, be, w2, b2):
    full = lambda shape: pl.BlockSpec(shape, lambda i: (0, 0))
    return pl.pallas_call(
        _q_body,
        grid=(N // NB,),
        in_specs=[
            pl.BlockSpec((NB, D), lambda i: (i, 0)),
            full((D, D)), full((1, D)), full((1, D)), full((1, D)),
            full((D, D)), full((1, D)),
        ],
        out_specs=pl.BlockSpec((NB, D), lambda i: (i, 0)),
        out_shape=jax.ShapeDtypeStruct((N, D), jnp.float32),
    )(h, w1, b1, g, be, w2, b2)


def _sc_gather_body(h_hbm, q_hbm, src_hbm, dst_hbm, hs_out, qd_out,
                    idxsrc, idxdst, hsrows, qdrows,
                    gh0, gh1, gq0, gq1, wh0, wh1, wq0, wq1):
    c = lax.axis_index("c")
    s = lax.axis_index("s")
    w = c * NS + s
    ebase = w * EPW

    pltpu.sync_copy(src_hbm.at[pl.ds(ebase, EPW)], idxsrc)
    pltpu.sync_copy(dst_hbm.at[pl.ds(ebase, EPW)], idxdst)

    ghs = (gh0, gh1)
    gqs = (gq0, gq1)
    whs = (wh0, wh1)
    wqs = (wq0, wq1)

    def pairbody(i, carry):
        gs = []
        for b in range(2):
            j = 2 * i + b
            off = j * CH
            gs.append(pltpu.async_copy(
                h_hbm.at[idxsrc.at[pl.ds(off, CH)]], hsrows.at[b], ghs[b]))
            gs.append(pltpu.async_copy(
                q_hbm.at[idxdst.at[pl.ds(off, CH)]], qdrows.at[b], gqs[b]))
        ws = []
        for b in range(2):
            j = 2 * i + b
            base = ebase + j * CH
            gs[2 * b].wait()
            ws.append(pltpu.async_copy(
                hsrows.at[b], hs_out.at[pl.ds(base, CH)], whs[b]))
            gs[2 * b + 1].wait()
            ws.append(pltpu.async_copy(
                qdrows.at[b], qd_out.at[pl.ds(base, CH)], wqs[b]))
        for wcp in ws:
            wcp.wait()
        return carry

    lax.fori_loop(0, NFULL // 2, pairbody, 0)

    toff = NFULL * CH
    tb = ebase + toff
    cp1 = pltpu.async_copy(
        h_hbm.at[idxsrc.at[pl.ds(toff, TAIL)]],
        hsrows.at[0, pl.ds(0, TAIL)], gh0)
    cp2 = pltpu.async_copy(
        q_hbm.at[idxdst.at[pl.ds(toff, TAIL)]],
        qdrows.at[0, pl.ds(0, TAIL)], gq0)
    cp1.wait()
    cp2.wait()
    pltpu.sync_copy(hsrows.at[0, pl.ds(0, TAIL)], hs_out.at[pl.ds(tb, TAIL)])
    pltpu.sync_copy(qdrows.at[0, pl.ds(0, TAIL)], qd_out.at[pl.ds(tb, TAIL)])


def _sc_gather(h, q, src, dst):
    mesh = plsc.VectorSubcoreMesh(core_axis_name="c", subcore_axis_name="s")
    f = functools.partial(
        pl.kernel,
        out_type=[
            jax.ShapeDtypeStruct((E, D), jnp.float32),
            jax.ShapeDtypeStruct((E, D), jnp.float32),
        ],
        mesh=mesh,
        scratch_types=[
            pltpu.VMEM((EPW,), jnp.int32),
            pltpu.VMEM((EPW,), jnp.int32),
            pltpu.VMEM((2, CH, D), jnp.float32),
            pltpu.VMEM((2, CH, D), jnp.float32),
        ] + [pltpu.SemaphoreType.DMA] * 8,
    )(_sc_gather_body)
    return f(h, q, src, dst)


def _edge_body(hs, qd, ea, kw1h, kw1e, kb1, kg, kbe, kw2, kb2,
               vw1h, vw1e, vb1, vg, vbe, vw2a, vw2b, vb2a, vb2b, s_ref,
               b0_ref, b1_ref, p0_ref, p1_ref):
    hsv = hs[...]
    eav = ea[...]
    tk = (jnp.dot(hsv, kw1h[...], preferred_element_type=jnp.float32)
          + jnp.dot(eav, kw1e[...], preferred_element_type=jnp.float32) + kb1[...])
    kk = jnp.dot(_ln_relu(tk, kg[...], kbe[...]), kw2[...],
                 preferred_element_type=jnp.float32) + kb2[...]
    tv = (jnp.dot(hsv, vw1h[...], preferred_element_type=jnp.float32)
          + jnp.dot(eav, vw1e[...], preferred_element_type=jnp.float32) + vb1[...])
    rv = _ln_relu(tv, vg[...], vbe[...])
    a0 = jnp.dot(rv, vw2a[...], preferred_element_type=jnp.float32) + vb2a[...]
    a1 = jnp.dot(rv, vw2b[...], preferred_element_type=jnp.float32) + vb2b[...]
    logits = jnp.dot(qd[...] * kk, s_ref[...],
                     preferred_element_type=jnp.float32) * (1.0 / np.sqrt(DH))
    ex = jnp.exp(logits)
    p0_ref[...] = a0 * jnp.dot(ex, b0_ref[...], preferred_element_type=jnp.float32)
    p1_ref[...] = a1 * jnp.dot(ex, b1_ref[...], preferred_element_type=jnp.float32)


def _tc_edge(hs, qd, ea, kw1h, kw1e, kb1, kg, kbe, kw2, kb2,
             vw1h, vw1e, vb1, vg, vbe, vw2a, vw2b, vb2a, vb2b,
             s_mat, b0_mat, b1_mat):
    full = lambda shape: pl.BlockSpec(shape, lambda i: (0, 0))
    return pl.pallas_call(
        _edge_body,
        grid=(E // EB,),
        in_specs=[
            pl.BlockSpec((EB, D), lambda i: (i, 0)),
            pl.BlockSpec((EB, D), lambda i: (i, 0)),
            pl.BlockSpec((EB, EF), lambda i: (i, 0)),
            full((D, D)), full((EF, D)), full((1, D)), full((1, D)), full((1, D)),
            full((D, D)), full((1, D)),
            full((D, D)), full((EF, D)), full((1, D)), full((1, D)), full((1, D)),
            full((D, D)), full((D, D)), full((1, D)), full((1, D)),
            full((D, H)), full((H, D)), full((H, D)),
        ],
        out_specs=[
            pl.BlockSpec((EB, D), lambda i: (i, 0)),
            pl.BlockSpec((EB, D), lambda i: (i, 0)),
        ],
        out_shape=[
            jax.ShapeDtypeStruct((E, D), jnp.float32),
            jax.ShapeDtypeStruct((E, D), jnp.float32),
        ],
    )(hs, qd, ea, kw1h, kw1e, kb1, kg, kbe, kw2, kb2,
      vw1h, vw1e, vb1, vg, vbe, vw2a, vw2b, vb2a, vb2b, s_mat, b0_mat, b1_mat)


def _sc_scatter_body(p0_hbm, p1_hbm, dst_hbm, zp_hbm, acc_out,
                     acc_sh, idxr, pbuf, idxt, pbuft, rbuf,
                     is0, is1, rs0, rs1, as0, as1):
    c = lax.axis_index("c")
    s = lax.axis_index("s")
    isems = (is0, is1)
    rsems = (rs0, rs1)
    asems = (as0, as1)

    nz_mine = (NZ - s + NS - 1) // NS
    pltpu.sync_copy(zp_hbm, rbuf)

    def zbody(i, carry):
        r0 = (s + i * NS) * ZCH
        pltpu.sync_copy(rbuf, acc_sh.at[pl.ds(r0, ZCH)])
        return carry

    lax.fori_loop(0, nz_mine, zbody, 0)
    plsc.subcore_barrier()

    abase = s * EPT2

    def run_acc(p_hbm):
        def pairbody(i, carry):
            cps = []
            for b in range(2):
                base = abase + (2 * i + b) * CH
                cps.append(pltpu.async_copy(
                    dst_hbm.at[pl.ds(base, CH)], idxr.at[b], isems[b]))
                cps.append(pltpu.async_copy(
                    p_hbm.at[pl.ds(base, CH)], pbuf.at[b], rsems[b]))
            adds = []
            for b in range(2):
                cps[2 * b].wait()
                cps[2 * b + 1].wait()
                adds.append(pltpu.async_copy(
                    pbuf.at[b], acc_sh.at[idxr.at[b]], asems[b], add=True))
            for a in adds:
                a.wait()
            return carry

        lax.fori_loop(0, NFULL2 // 2, pairbody, 0)

        base = abase + NFULL2 * CH
        pltpu.sync_copy(dst_hbm.at[pl.ds(base, TAIL2)], idxt)
        pltpu.sync_copy(p_hbm.at[pl.ds(base, TAIL2)], pbuft)
        pltpu.sync_copy(pbuft, acc_sh.at[idxt], add=True)

    @pl.when(c == 0)
    def _():
        run_acc(p0_hbm)

    @pl.when(c == 1)
    def _():
        run_acc(p1_hbm)

    plsc.subcore_barrier()

    def fbody(i, carry):
        r0 = (s + i * NS) * ZCH
        pltpu.sync_copy(acc_sh.at[pl.ds(r0, ZCH)], rbuf)
        pltpu.sync_copy(rbuf, acc_out.at[c, pl.ds(r0, ZCH)])
        return carry

    lax.fori_loop(0, nz_mine, fbody, 0)


def _sc_scatter(p0, p1, dst, zp):
    mesh = plsc.VectorSubcoreMesh(core_axis_name="c", subcore_axis_name="s")
    f = functools.partial(
        pl.kernel,
        out_type=jax.ShapeDtypeStruct((NC, N, D), jnp.float32),
        mesh=mesh,
        scratch_types=[
            pltpu.VMEM_SHARED((N, D), jnp.float32),
            pltpu.VMEM((2, CH), jnp.int32),
            pltpu.VMEM((2, CH, D), jnp.float32),
            pltpu.VMEM((TAIL2,), jnp.int32),
            pltpu.VMEM((TAIL2, D), jnp.float32),
            pltpu.VMEM((ZCH, D), jnp.float32),
        ] + [pltpu.SemaphoreType.DMA] * 6,
    )(_sc_scatter_body)
    return f(p0, p1, dst, zp)


def _out_body(a0, a1, d0, d1, h_ref, b0_ref, b1_ref,
              w1a0, w1a1, w1b, b1, g, be, w2, b2, o_ref):
    oa0 = a0[...] / (jnp.dot(d0[...], b0_ref[...], preferred_element_type=jnp.float32) + 1e-16)
    oa1 = a1[...] / (jnp.dot(d1[...], b1_ref[...], preferred_element_type=jnp.float32) + 1e-16)
    hv = h_ref[...]
    t = (jnp.dot(oa0, w1a0[...], preferred_element_type=jnp.float32)
         + jnp.dot(oa1, w1a1[...], preferred_element_type=jnp.float32)
         + jnp.dot(hv, w1b[...], preferred_element_type=jnp.float32) + b1[...])
    t = _ln_relu(t, g[...], be[...])
    o_ref[...] = jnp.dot(t, w2[...], preferred_element_type=jnp.float32) + b2[...] + hv


def _tc_out(a0, a1, d0, d1, h, b0_mat, b1_mat, w1a0, w1a1, w1b, b1, g, be, w2, b2):
    full = lambda shape: pl.BlockSpec(shape, lambda i: (0, 0))
    return pl.pallas_call(
        _out_body,
        grid=(N // NB,),
        in_specs=[
            pl.BlockSpec((NB, D // 2), lambda i: (i, 0)),
            pl.BlockSpec((NB, D // 2), lambda i: (i, 0)),
            pl.BlockSpec((NB, H), lambda i: (i, 0)),
            pl.BlockSpec((NB, H), lambda i: (i, 0)),
            pl.BlockSpec((NB, D), lambda i: (i, 0)),
            full((H, D // 2)), full((H, D // 2)),
            full((D // 2, D)), full((D // 2, D)), full((D, D)),
            full((1, D)), full((1, D)), full((1, D)),
            full((D, D)), full((1, D)),
        ],
        out_specs=pl.BlockSpec((NB, D), lambda i: (i, 0)),
        out_shape=jax.ShapeDtypeStruct((N, D), jnp.float32),
    )(a0, a1, d0, d1, h, b0_mat, b1_mat, w1a0, w1a1, w1b, b1, g, be, w2, b2)


def kernel(h, edge_attr, edge_index,
           hk_W1, hk_b1, hk_g, hk_be, hk_W2, hk_b2,
           hv_W1, hv_b1, hv_g, hv_be, hv_W2, hv_b2,
           hq_W1, hq_b1, hq_g, hq_be, hq_W2, hq_b2,
           no_W1, no_b1, no_g, no_be, no_W2, no_b2):
    src = edge_index[0]
    dst = edge_index[1]
    r = lambda x: x.reshape(1, D)

    s_mat = jnp.asarray(_S_NP)
    b0_mat = jnp.asarray(_B0_NP)
    b1_mat = jnp.asarray(_B1_NP)
    b0e_mat = jnp.asarray(_B0E_NP)
    b1e_mat = jnp.asarray(_B1E_NP)
    half = D // 2
    zcol = jnp.zeros((D, half), jnp.float32)
    ebias = jnp.concatenate([jnp.ones((H,), jnp.float32),
                             jnp.zeros((half - H,), jnp.float32)])
    vw2a = jnp.concatenate([hv_W2[:, :half], zcol], axis=1)
    vw2b = jnp.concatenate([hv_W2[:, half:], zcol], axis=1)
    vb2a = jnp.concatenate([hv_b2[:half], ebias]).reshape(1, D)
    vb2b = jnp.concatenate([hv_b2[half:], ebias]).reshape(1, D)

    q = _tc_q(h, hq_W1, r(hq_b1), r(hq_g), r(hq_be), hq_W2, r(hq_b2))
    hs, qd = _sc_gather(h, q, src, dst)
    p0, p1 = _tc_edge(
        hs, qd, edge_attr,
        hk_W1[:D], hk_W1[D:], r(hk_b1), r(hk_g), r(hk_be), hk_W2, r(hk_b2),
        hv_W1[:D], hv_W1[D:], r(hv_b1), r(hv_g), r(hv_be),
        vw2a, vw2b, vb2a, vb2b,
        s_mat, b0e_mat, b1e_mat)
    zp = jnp.zeros((ZCH, D), jnp.float32)
    acc = _sc_scatter(p0, p1, dst, zp)
    a0 = acc[0, :, :half]
    d0 = acc[0, :, half:half + H]
    a1 = acc[1, :, :half]
    d1 = acc[1, :, half:half + H]
    out = _tc_out(a0, a1, d0, d1, h, b0_mat, b1_mat,
                  no_W1[:half], no_W1[half:D], no_W1[D:],
                  r(no_b1), r(no_g), r(no_be), no_W2, r(no_b2))
    return out


# speedup vs baseline: 45.8911x; 1.1454x over previous
import functools

import jax
import jax.numpy as jnp
import numpy as np
from jax import lax
from jax.experimental import pallas as pl
from jax.experimental.pallas import tpu as pltpu
from jax.experimental.pallas import tpu_sc as plsc

N = 10000
E = 320000
D = 128
H = 16
EF = 16
DH = D // H

NC = 2
NS = 16
NW = NC * NS
EPW = E // NW
CH = 128
NFULL = EPW // CH
TAIL = EPW - NFULL * CH
ZCH = 80
NZ = N // ZCH
EPT2 = 2 * EPW
NFULL2 = EPT2 // CH
TAIL2 = EPT2 - NFULL2 * CH

_S_NP = np.kron(np.eye(H, dtype=np.float32), np.ones((DH, 1), dtype=np.float32))
_B8 = np.kron(np.eye(H // 2, dtype=np.float32), np.ones((1, DH), dtype=np.float32))
_B0_NP = np.vstack([_B8, np.zeros_like(_B8)])
_B1_NP = np.vstack([np.zeros_like(_B8), _B8])
_B0E_NP = np.hstack([_B0_NP, np.eye(H, dtype=np.float32),
                     np.zeros((H, D - D // 2 - H), np.float32)])
_B1E_NP = np.hstack([_B1_NP, np.eye(H, dtype=np.float32),
                     np.zeros((H, D - D // 2 - H), np.float32)])

EB = 2000
NB = 2000

EHALF = E // 2
EPWH = EHALF // NW
NFULLH = EPWH // CH
TAILH = EPWH - NFULLH * CH
NPAIRH = NFULLH // 2
EPTH = EHALF // NS
NF2H = EPTH // CH
TAIL2H = EPTH - NF2H * CH


def _ln_relu(t, g, be):
    mu = jnp.mean(t, axis=-1, keepdims=True)
    var = jnp.mean((t - mu) * (t - mu), axis=-1, keepdims=True)
    return jax.nn.relu((t - mu) * lax.rsqrt(var + 1e-5) * g + be)


def _q_body(h_ref, w1, b1, g, be, w2, b2, q_ref):
    t = jnp.dot(h_ref[...], w1[...], preferred_element_type=jnp.float32) + b1[...]
    t = _ln_relu(t, g[...], be[...])
    q_ref[...] = jnp.dot(t, w2[...], preferred_element_type=jnp.float32) + b2[...]


def _tc_q(h, w1, b1, g, be, w2, b2):
    full = lambda shape: pl.BlockSpec(shape, lambda i: (0, 0))
    return pl.pallas_call(
        _q_body,
        grid=(N // NB,),
        in_specs=[
            pl.BlockSpec((NB, D), lambda i: (i, 0)),
            full((D, D)), full((1, D)), full((1, D)), full((1, D)),
            full((D, D)), full((1, D)),
        ],
        out_specs=pl.BlockSpec((NB, D), lambda i: (i, 0)),
        out_shape=jax.ShapeDtypeStruct((N, D), jnp.float32),
    )(h, w1, b1, g, be, w2, b2)


def _make_gather_body(e0):
    def body(h_hbm, q_hbm, src_hbm, dst_hbm, hs_out, qd_out,
             idxsrc, idxdst, hsrows, qdrows,
             gh0, gh1, gq0, gq1, wh0, wh1, wq0, wq1):
        c = lax.axis_index("c")
        s = lax.axis_index("s")
        w = c * NS + s
        obase = w * EPWH
        ebase = e0 + obase

        pltpu.sync_copy(src_hbm.at[pl.ds(ebase, EPWH)], idxsrc)
        pltpu.sync_copy(dst_hbm.at[pl.ds(ebase, EPWH)], idxdst)

        ghs = (gh0, gh1)
        gqs = (gq0, gq1)
        whs = (wh0, wh1)
        wqs = (wq0, wq1)

        def do_chunk_pair(i, npair):
            gs = []
            for b in range(npair):
                off = (2 * i + b) * CH
                gs.append(pltpu.async_copy(
                    h_hbm.at[idxsrc.at[pl.ds(off, CH)]], hsrows.at[b], ghs[b]))
                gs.append(pltpu.async_copy(
                    q_hbm.at[idxdst.at[pl.ds(off, CH)]], qdrows.at[b], gqs[b]))
            ws = []
            for b in range(npair):
                base = obase + (2 * i + b) * CH
                gs[2 * b].wait()
                ws.append(pltpu.async_copy(
                    hsrows.at[b], hs_out.at[pl.ds(base, CH)], whs[b]))
                gs[2 * b + 1].wait()
                ws.append(pltpu.async_copy(
                    qdrows.at[b], qd_out.at[pl.ds(base, CH)], wqs[b]))
            for wcp in ws:
                wcp.wait()

        def pairbody(i, carry):
            do_chunk_pair(i, 2)
            return carry

        lax.fori_loop(0, NPAIRH, pairbody, 0)
        if NFULLH % 2:
            do_chunk_pair(NFULLH // 2, 1)

        toff = NFULLH * CH
        tb = obase + toff
        cp1 = pltpu.async_copy(
            h_hbm.at[idxsrc.at[pl.ds(toff, TAILH)]],
            hsrows.at[0, pl.ds(0, TAILH)], gh0)
        cp2 = pltpu.async_copy(
            q_hbm.at[idxdst.at[pl.ds(toff, TAILH)]],
            qdrows.at[0, pl.ds(0, TAILH)], gq0)
        cp1.wait()
        cp2.wait()
        pltpu.sync_copy(hsrows.at[0, pl.ds(0, TAILH)],
                        hs_out.at[pl.ds(tb, TAILH)])
        pltpu.sync_copy(qdrows.at[0, pl.ds(0, TAILH)],
                        qd_out.at[pl.ds(tb, TAILH)])

    return body


def _sc_gather_half(h, q, src, dst, e0):
    mesh = plsc.VectorSubcoreMesh(core_axis_name="c", subcore_axis_name="s")
    f = functools.partial(
        pl.kernel,
        out_type=[
            jax.ShapeDtypeStruct((EHALF, D), jnp.float32),
            jax.ShapeDtypeStruct((EHALF, D), jnp.float32),
        ],
        mesh=mesh,
        scratch_types=[
            pltpu.VMEM((EPWH,), jnp.int32),
            pltpu.VMEM((EPWH,), jnp.int32),
            pltpu.VMEM((2, CH, D), jnp.float32),
            pltpu.VMEM((2, CH, D), jnp.float32),
        ] + [pltpu.SemaphoreType.DMA] * 8,
    )(_make_gather_body(e0))
    return f(h, q, src, dst)


def _edge_body(hs, qd, ea, kw1h, kw1e, kb1, kg, kbe, kw2, kb2,
               vw1h, vw1e, vb1, vg, vbe, vw2a, vw2b, vb2a, vb2b, s_ref,
               b0_ref, b1_ref, p0_ref, p1_ref):
    hsv = hs[...]
    eav = ea[...]
    tk = (jnp.dot(hsv, kw1h[...], preferred_element_type=jnp.float32)
          + jnp.dot(eav, kw1e[...], preferred_element_type=jnp.float32) + kb1[...])
    kk = jnp.dot(_ln_relu(tk, kg[...], kbe[...]), kw2[...],
                 preferred_element_type=jnp.float32) + kb2[...]
    tv = (jnp.dot(hsv, vw1h[...], preferred_element_type=jnp.float32)
          + jnp.dot(eav, vw1e[...], preferred_element_type=jnp.float32) + vb1[...])
    rv = _ln_relu(tv, vg[...], vbe[...])
    a0 = jnp.dot(rv, vw2a[...], preferred_element_type=jnp.float32) + vb2a[...]
    a1 = jnp.dot(rv, vw2b[...], preferred_element_type=jnp.float32) + vb2b[...]
    logits = jnp.dot(qd[...] * kk, s_ref[...],
                     preferred_element_type=jnp.float32) * (1.0 / np.sqrt(DH))
    ex = jnp.exp(logits)
    p0_ref[...] = a0 * jnp.dot(ex, b0_ref[...], preferred_element_type=jnp.float32)
    p1_ref[...] = a1 * jnp.dot(ex, b1_ref[...], preferred_element_type=jnp.float32)


def _tc_edge(hs, qd, ea, kw1h, kw1e, kb1, kg, kbe, kw2, kb2,
             vw1h, vw1e, vb1, vg, vbe, vw2a, vw2b, vb2a, vb2b,
             s_mat, b0_mat, b1_mat, eoff):
    full = lambda shape: pl.BlockSpec(shape, lambda i: (0, 0))
    boff = eoff // EB
    return pl.pallas_call(
        _edge_body,
        grid=(EHALF // EB,),
        in_specs=[
            pl.BlockSpec((EB, D), lambda i: (i, 0)),
            pl.BlockSpec((EB, D), lambda i: (i, 0)),
            pl.BlockSpec((EB, EF), lambda i: (i + boff, 0)),
            full((D, D)), full((EF, D)), full((1, D)), full((1, D)), full((1, D)),
            full((D, D)), full((1, D)),
            full((D, D)), full((EF, D)), full((1, D)), full((1, D)), full((1, D)),
            full((D, D)), full((D, D)), full((1, D)), full((1, D)),
            full((D, H)), full((H, D)), full((H, D)),
        ],
        out_specs=[
            pl.BlockSpec((EB, D), lambda i: (i, 0)),
            pl.BlockSpec((EB, D), lambda i: (i, 0)),
        ],
        out_shape=[
            jax.ShapeDtypeStruct((EHALF, D), jnp.float32),
            jax.ShapeDtypeStruct((EHALF, D), jnp.float32),
        ],
    )(hs, qd, ea, kw1h, kw1e, kb1, kg, kbe, kw2, kb2,
      vw1h, vw1e, vb1, vg, vbe, vw2a, vw2b, vb2a, vb2b, s_mat, b0_mat, b1_mat)


def _make_scatter_body(e0):
    def body(p0_hbm, p1_hbm, dst_hbm, zp_hbm, acc_out,
             acc_sh, idxr, pbuf, idxt, pbuft, rbuf,
             is0, is1, rs0, rs1, as0, as1):
        c = lax.axis_index("c")
        s = lax.axis_index("s")
        isems = (is0, is1)
        rsems = (rs0, rs1)
        asems = (as0, as1)

        nz_mine = (NZ - s + NS - 1) // NS
        pltpu.sync_copy(zp_hbm, rbuf)

        def zbody(i, carry):
            r0 = (s + i * NS) * ZCH
            pltpu.sync_copy(rbuf, acc_sh.at[pl.ds(r0, ZCH)])
            return carry

        lax.fori_loop(0, nz_mine, zbody, 0)
        plsc.subcore_barrier()

        obase = s * EPTH
        dbase = e0 + obase

        def run_acc(p_hbm):
            def pairbody(i, carry):
                cps = []
                for b in range(2):
                    off = (2 * i + b) * CH
                    cps.append(pltpu.async_copy(
                        dst_hbm.at[pl.ds(dbase + off, CH)], idxr.at[b],
                        isems[b]))
                    cps.append(pltpu.async_copy(
                        p_hbm.at[pl.ds(obase + off, CH)], pbuf.at[b],
                        rsems[b]))
                adds = []
                for b in range(2):
                    cps[2 * b].wait()
                    cps[2 * b + 1].wait()
                    adds.append(pltpu.async_copy(
                        pbuf.at[b], acc_sh.at[idxr.at[b]], asems[b],
                        add=True))
                for a in adds:
                    a.wait()
                return carry

            lax.fori_loop(0, NF2H // 2, pairbody, 0)

            toff = NF2H * CH
            pltpu.sync_copy(dst_hbm.at[pl.ds(dbase + toff, TAIL2H)], idxt)
            pltpu.sync_copy(p_hbm.at[pl.ds(obase + toff, TAIL2H)], pbuft)
            pltpu.sync_copy(pbuft, acc_sh.at[idxt], add=True)

        @pl.when(c == 0)
        def _():
            run_acc(p0_hbm)

        @pl.when(c == 1)
        def _():
            run_acc(p1_hbm)

        plsc.subcore_barrier()

        def fbody(i, carry):
            r0 = (s + i * NS) * ZCH
            pltpu.sync_copy(acc_sh.at[pl.ds(r0, ZCH)], rbuf)
            pltpu.sync_copy(rbuf, acc_out.at[c, pl.ds(r0, ZCH)])
            return carry

        lax.fori_loop(0, nz_mine, fbody, 0)

    return body


def _sc_scatter_half(p0, p1, dst, zp, e0):
    mesh = plsc.VectorSubcoreMesh(core_axis_name="c", subcore_axis_name="s")
    f = functools.partial(
        pl.kernel,
        out_type=jax.ShapeDtypeStruct((NC, N, D), jnp.float32),
        mesh=mesh,
        scratch_types=[
            pltpu.VMEM_SHARED((N, D), jnp.float32),
            pltpu.VMEM((2, CH), jnp.int32),
            pltpu.VMEM((2, CH, D), jnp.float32),
            pltpu.VMEM((TAIL2H,), jnp.int32),
            pltpu.VMEM((TAIL2H, D), jnp.float32),
            pltpu.VMEM((ZCH, D), jnp.float32),
        ] + [pltpu.SemaphoreType.DMA] * 6,
    )(_make_scatter_body(e0))
    return f(p0, p1, dst, zp)


def _out_body(a0x, a0y, a1x, a1y, d0x, d0y, d1x, d1y, h_ref, b0_ref, b1_ref,
              w1a0, w1a1, w1b, b1, g, be, w2, b2, o_ref):
    a0 = a0x[...] + a0y[...]
    a1 = a1x[...] + a1y[...]
    d0 = d0x[...] + d0y[...]
    d1 = d1x[...] + d1y[...]
    oa0 = a0 / (jnp.dot(d0, b0_ref[...], preferred_element_type=jnp.float32) + 1e-16)
    oa1 = a1 / (jnp.dot(d1, b1_ref[...], preferred_element_type=jnp.float32) + 1e-16)
    hv = h_ref[...]
    t = (jnp.dot(oa0, w1a0[...], preferred_element_type=jnp.float32)
         + jnp.dot(oa1, w1a1[...], preferred_element_type=jnp.float32)
         + jnp.dot(hv, w1b[...], preferred_element_type=jnp.float32) + b1[...])
    t = _ln_relu(t, g[...], be[...])
    o_ref[...] = jnp.dot(t, w2[...], preferred_element_type=jnp.float32) + b2[...] + hv


def _tc_out(a0x, a0y, a1x, a1y, d0x, d0y, d1x, d1y, h,
            b0_mat, b1_mat, w1a0, w1a1, w1b, b1, g, be, w2, b2):
    full = lambda shape: pl.BlockSpec(shape, lambda i: (0, 0))
    rowsD2 = pl.BlockSpec((NB, D // 2), lambda i: (i, 0))
    rowsH = pl.BlockSpec((NB, H), lambda i: (i, 0))
    return pl.pallas_call(
        _out_body,
        grid=(N // NB,),
        in_specs=[
            rowsD2, rowsD2, rowsD2, rowsD2,
            rowsH, rowsH, rowsH, rowsH,
            pl.BlockSpec((NB, D), lambda i: (i, 0)),
            full((H, D // 2)), full((H, D // 2)),
            full((D // 2, D)), full((D // 2, D)), full((D, D)),
            full((1, D)), full((1, D)), full((1, D)),
            full((D, D)), full((1, D)),
        ],
        out_specs=pl.BlockSpec((NB, D), lambda i: (i, 0)),
        out_shape=jax.ShapeDtypeStruct((N, D), jnp.float32),
    )(a0x, a0y, a1x, a1y, d0x, d0y, d1x, d1y, h,
      b0_mat, b1_mat, w1a0, w1a1, w1b, b1, g, be, w2, b2)


def kernel(h, edge_attr, edge_index,
           hk_W1, hk_b1, hk_g, hk_be, hk_W2, hk_b2,
           hv_W1, hv_b1, hv_g, hv_be, hv_W2, hv_b2,
           hq_W1, hq_b1, hq_g, hq_be, hq_W2, hq_b2,
           no_W1, no_b1, no_g, no_be, no_W2, no_b2):
    src = edge_index[0]
    dst = edge_index[1]
    r = lambda x: x.reshape(1, D)

    s_mat = jnp.asarray(_S_NP)
    b0_mat = jnp.asarray(_B0_NP)
    b1_mat = jnp.asarray(_B1_NP)
    b0e_mat = jnp.asarray(_B0E_NP)
    b1e_mat = jnp.asarray(_B1E_NP)
    half = D // 2
    zcol = jnp.zeros((D, half), jnp.float32)
    ebias = jnp.concatenate([jnp.ones((H,), jnp.float32),
                             jnp.zeros((half - H,), jnp.float32)])
    vw2a = jnp.concatenate([hv_W2[:, :half], zcol], axis=1)
    vw2b = jnp.concatenate([hv_W2[:, half:], zcol], axis=1)
    vb2a = jnp.concatenate([hv_b2[:half], ebias]).reshape(1, D)
    vb2b = jnp.concatenate([hv_b2[half:], ebias]).reshape(1, D)

    q = _tc_q(h, hq_W1, r(hq_b1), r(hq_g), r(hq_be), hq_W2, r(hq_b2))
    zp = jnp.zeros((ZCH, D), jnp.float32)
    ew = (hk_W1[:D], hk_W1[D:], r(hk_b1), r(hk_g), r(hk_be), hk_W2, r(hk_b2),
          hv_W1[:D], hv_W1[D:], r(hv_b1), r(hv_g), r(hv_be),
          vw2a, vw2b, vb2a, vb2b, s_mat, b0e_mat, b1e_mat)

    hsA, qdA = _sc_gather_half(h, q, src, dst, 0)
    hsB, qdB = _sc_gather_half(h, q, src, dst, EHALF)
    p0A, p1A = _tc_edge(hsA, qdA, edge_attr, *ew, eoff=0)
    p0B, p1B = _tc_edge(hsB, qdB, edge_attr, *ew, eoff=EHALF)
    accA = _sc_scatter_half(p0A, p1A, dst, zp, 0)
    accB = _sc_scatter_half(p0B, p1B, dst, zp, EHALF)

    out = _tc_out(accA[0, :, :half], accB[0, :, :half],
                  accA[1, :, :half], accB[1, :, :half],
                  accA[0, :, half:half + H], accB[0, :, half:half + H],
                  accA[1, :, half:half + H], accB[1, :, half:half + H],
                  h, b0_mat, b1_mat,
                  no_W1[:half], no_W1[half:D], no_W1[D:],
                  r(no_b1), r(no_g), r(no_be), no_W2, r(no_b2))
    return out
